# Initial kernel scaffold; baseline (speedup 1.0000x reference)
import jax
import jax.numpy as jnp
from jax.experimental import pallas as pl


def kernel(logprobs, beam_seq, beam_seq_logprobs, beam_logprobs_sum, state, t):
    raise NotImplementedError("write your pallas kernel here")


# speedup vs baseline: 14.8619x; 14.8619x over previous
import functools

import jax
import jax.numpy as jnp
from jax import lax
from jax.experimental import pallas as pl
from jax.experimental.pallas import tpu as pltpu
from jax.experimental.pallas import tpu_sc as plsc

B, BEAM, V, L, H, LAYERS = 64, 5, 100000, 20, 512, 2
UNK = 3
LP = 32
ROWS = B * BEAM
NC, NS = 2, 16
NW = NC * NS
BPW = B // NW
RPW = BPW * BEAM
SIDX = LAYERS * RPW
CH = 20000
NCH = V // CH
GRP = CH // 16
NEG = -3.0e38
IMAX = 2**31 - 1


def _topk_row(lp_hbm, chunk, row):
    iota = lax.iota(jnp.int32, 16)

    def chunk_body(c, carry):
        off = pl.multiple_of(c * CH, CH)
        pltpu.sync_copy(lp_hbm.at[row, pl.ds(off, CH)], chunk)

        @pl.when(c == 0)
        def _():
            g = chunk[pl.ds(0, 16)]
            chunk[pl.ds(0, 16)] = g - jnp.where(
                iota == UNK, jnp.float32(1000.0), jnp.float32(0.0))

        def group_body(g, cr):
            ms = list(cr[:5])
            js = list(cr[5:])
            goff = pl.multiple_of(g * 16, 16)
            v = chunk[pl.ds(goff, 16)]
            iv = iota + (c * CH + g * 16)
            for k in range(5):
                gt = v > ms[k]
                nm = jnp.where(gt, v, ms[k])
                nj = jnp.where(gt, iv, js[k])
                if k < 4:
                    nv = jnp.where(gt, ms[k], v)
                    niv = jnp.where(gt, js[k], iv)
                    v, iv = nv, niv
                ms[k], js[k] = nm, nj
            return (*ms, *js)

        return lax.fori_loop(0, GRP, group_body, carry)

    neg = jnp.full((16,), NEG, jnp.float32)
    zero = jnp.zeros((16,), jnp.int32)
    init = (neg, neg, neg, neg, neg, zero, zero, zero, zero, zero)
    return lax.fori_loop(0, NCH, chunk_body, init)


def _merge5(ms, js, iota):
    valvec = jnp.full((16,), NEG, jnp.float32)
    idxvec = jnp.zeros((16,), jnp.int32)
    for it in range(5):
        mv = jnp.max(ms[0])
        eq = ms[0] == mv
        mi = jnp.min(jnp.where(eq, js[0], IMAX))
        lane = eq & (js[0] == mi)
        valvec = jnp.where(iota == it, mv, valvec)
        idxvec = jnp.where(iota == it, mi, idxvec)
        for k in range(4):
            ms[k] = jnp.where(lane, ms[k + 1], ms[k])
            js[k] = jnp.where(lane, js[k + 1], js[k])
        ms[4] = jnp.where(lane, NEG, ms[4])
        js[4] = jnp.where(lane, 0, js[4])
    return valvec, idxvec


_MESH = plsc.VectorSubcoreMesh(
    core_axis_name="c", subcore_axis_name="s", num_cores=NC, num_subcores=NS)


@functools.partial(
    pl.kernel,
    out_type=(
        jax.ShapeDtypeStruct((ROWS, LP), jnp.int32),
        jax.ShapeDtypeStruct((ROWS, LP), jnp.float32),
        jax.ShapeDtypeStruct((B, 16), jnp.float32),
        jax.ShapeDtypeStruct((LAYERS * ROWS, H), jnp.float32),
    ),
    mesh=_MESH,
    compiler_params=pltpu.CompilerParams(
        use_tc_tiling_on_sc=False, needs_layout_passes=False),
    scratch_types=[
        pltpu.VMEM((CH,), jnp.float32),
        pltpu.VMEM((16, 16), jnp.float32),
        pltpu.VMEM((16, 16), jnp.int32),
        pltpu.VMEM((BPW, 8), jnp.float32),
        pltpu.VMEM((16,), jnp.int32),
        pltpu.VMEM((BPW, 16), jnp.float32),
        pltpu.VMEM((RPW,), jnp.int32),
        pltpu.VMEM((RPW,), jnp.int32),
        pltpu.VMEM((SIDX,), jnp.int32),
        pltpu.VMEM((SIDX,), jnp.int32),
        pltpu.VMEM((RPW, LP), jnp.int32),
        pltpu.VMEM((RPW, LP), jnp.float32),
        pltpu.VMEM((SIDX, H), jnp.float32),
        pltpu.SemaphoreType.DMA,
    ],
)
def _beam_step(lp_hbm, seq_hbm, slp_hbm, sums_hbm, st_hbm, t_hbm,
               seq_out, slp_out, tsum_out, st_out,
               chunk, topv, topi, sums_vb, tvb, ts,
               gidx, oidx, sidx, soidx, seqb, slpb, stb, sem):
    wid = lax.axis_index("c") * NS + lax.axis_index("s")
    b0 = wid * BPW
    iota = lax.iota(jnp.int32, 16)
    negv = jnp.full((16,), NEG, jnp.float32)

    pltpu.sync_copy(t_hbm, tvb)
    pltpu.sync_copy(sums_hbm.at[pl.ds(b0, BPW)], sums_vb)

    for rr in range(RPW, 16):
        topv[rr, :] = negv
    for bi in range(BPW):
        for r in range(BEAM):
            row = (b0 + bi) * BEAM + r
            carry = _topk_row(lp_hbm, chunk, row)
            valvec, idxvec = _merge5(list(carry[:5]), list(carry[5:]), iota)
            topv[bi * BEAM + r, :] = valvec
            topi[bi * BEAM + r, :] = idxvec

    tok_all, slp_all = [], []
    gv = jnp.zeros((16,), jnp.int32)
    ov = jnp.zeros((16,), jnp.int32)
    sv0 = jnp.zeros((16,), jnp.int32)
    sv1 = jnp.zeros((16,), jnp.int32)
    so0 = jnp.zeros((16,), jnp.int32)
    so1 = jnp.zeros((16,), jnp.int32)
    for bi in range(BPW):
        b = b0 + bi
        bsplat = jnp.full((16,), bi, jnp.int32)
        r_lo, c_lo = iota // 5, iota % 5
        r_hi, c_hi = (iota + 16) // 5, (iota + 16) % 5
        base = bi * BEAM
        su0 = plsc.load_gather(sums_vb, [bsplat, r_lo])
        su1 = plsc.load_gather(sums_vb, [bsplat, r_hi])
        row_hi = jnp.where(iota + 16 < BEAM * BEAM, base + r_hi, 15)
        c0 = plsc.load_gather(topv, [base + r_lo, c_lo]) + su0
        c1 = plsc.load_gather(topv, [row_hi, c_hi]) + su1
        tsvec = jnp.zeros((16,), jnp.float32)
        for it in range(5):
            mv = jnp.maximum(jnp.max(c0), jnp.max(c1))
            p0 = jnp.min(jnp.where(c0 == mv, iota, IMAX))
            p1 = jnp.min(jnp.where(c1 == mv, iota + 16, IMAX))
            pos = jnp.minimum(p0, p1)
            tsvec = jnp.where(iota == it, mv, tsvec)
            rowsp = jnp.broadcast_to(base + pos // 5, (16,))
            colsp = jnp.broadcast_to(pos % 5, (16,))
            tok_all.append(plsc.load_gather(topi, [rowsp, colsp]))
            slp_all.append(plsc.load_gather(topv, [rowsp, colsp]))
            c0 = jnp.where(iota == pos, NEG, c0)
            c1 = jnp.where(iota + 16 == pos, NEG, c1)
            srow = b * BEAM + pos // 5
            drow = b * BEAM + it
            p = bi * BEAM + it
            gv = jnp.where(iota == p, srow, gv)
            ov = jnp.where(iota == p, drow, ov)
            for l in range(LAYERS):
                q = l * RPW + p
                sv0 = jnp.where(iota == q, l * ROWS + srow, sv0)
                sv1 = jnp.where(iota + 16 == q, l * ROWS + srow, sv1)
                so0 = jnp.where(iota == q, l * ROWS + drow, so0)
                so1 = jnp.where(iota + 16 == q, l * ROWS + drow, so1)
        ts[bi, :] = tsvec

    lo_mask = iota < RPW
    plsc.store_scatter(gidx, [iota], gv, mask=lo_mask)
    plsc.store_scatter(oidx, [iota], ov, mask=lo_mask)
    plsc.store_scatter(sidx, [iota], sv0)
    plsc.store_scatter(soidx, [iota], so0)
    hi_mask = iota < (SIDX - 16)
    plsc.store_scatter(sidx, [iota + 16], sv1, mask=hi_mask)
    plsc.store_scatter(soidx, [iota + 16], so1, mask=hi_mask)

    pltpu.async_copy(seq_hbm.at[gidx], seqb, sem).wait()
    pltpu.async_copy(slp_hbm.at[gidx], slpb, sem).wait()
    pltpu.async_copy(st_hbm.at[sidx], stb, sem).wait()

    tv = tvb[...]
    for k in range(RPW):
        g0 = seqb[k, pl.ds(0, 16)]
        seqb[k, pl.ds(0, 16)] = jnp.where(iota == tv, tok_all[k], g0)
        g1 = seqb[k, pl.ds(16, 16)]
        seqb[k, pl.ds(16, 16)] = jnp.where(iota + 16 == tv, tok_all[k], g1)
        f0 = slpb[k, pl.ds(0, 16)]
        slpb[k, pl.ds(0, 16)] = jnp.where(iota == tv, slp_all[k], f0)
        f1 = slpb[k, pl.ds(16, 16)]
        slpb[k, pl.ds(16, 16)] = jnp.where(iota + 16 == tv, slp_all[k], f1)

    pltpu.async_copy(seqb, seq_out.at[oidx], sem).wait()
    pltpu.async_copy(slpb, slp_out.at[oidx], sem).wait()
    pltpu.async_copy(stb, st_out.at[soidx], sem).wait()
    pltpu.sync_copy(ts, tsum_out.at[pl.ds(b0, BPW)])


def kernel(logprobs, beam_seq, beam_seq_logprobs, beam_logprobs_sum, state, t):
    lp2 = logprobs.reshape(ROWS, V)
    seq_p = jnp.pad(beam_seq.reshape(ROWS, L), ((0, 0), (0, LP - L)))
    slp_p = jnp.pad(beam_seq_logprobs.reshape(ROWS, L), ((0, 0), (0, LP - L)))
    sums_p = jnp.pad(beam_logprobs_sum, ((0, 0), (0, 8 - BEAM)))
    st2 = state.reshape(LAYERS * ROWS, H)
    tvec = jnp.full((16,), t, jnp.int32)
    seq_o, slp_o, tsum_o, st_o = _beam_step(lp2, seq_p, slp_p, sums_p, st2, tvec)
    new_seq = seq_o[:, :L].reshape(B, BEAM, L)
    new_seq_lp = slp_o[:, :L].reshape(B, BEAM, L)
    top_sums = tsum_o[:, :BEAM]
    new_state = st_o.reshape(LAYERS, B, BEAM, H)
    return (new_seq, new_seq_lp, top_sums, new_state)
